# Initial kernel scaffold; baseline (speedup 1.0000x reference)
import jax
import jax.numpy as jnp
from jax.experimental import pallas as pl


def kernel(x, edge_index, Wl1, bl1, Wr1, Wl2, bl2, Wr2, Wl3, bl3, Wr3):
    raise NotImplementedError("write your pallas kernel here")


# speedup vs baseline: 4.7008x; 4.7008x over previous
import functools

import jax
import jax.numpy as jnp
from jax import lax
from jax.experimental import pallas as pl
from jax.experimental.pallas import tpu as pltpu
from jax.experimental.pallas import tpu_sc as plsc

_N = 10000
_E = 320000
_D = 128
_NC = 2
_NS = 16
_NW = _NC * _NS
_EPW = _E // _NW
_K = 80
_NIT = _EPW // _K
_ZT = 10
_RPT = _N // _ZT
_CNT_T = 5
_CNT_R = _N // _CNT_T


def _sc_body(with_cnt, *refs):
    if with_cnt:
        (x_hbm, src_hbm, dst_hbm, z2_hbm,
         agg_hbm, cnt_hbm,
         acc_sh, cnt_sh, src_v, dst_v, rows_v, ones_v, cnt_v, sem) = refs
    else:
        (x_hbm, src_hbm, dst_hbm, z2_hbm,
         agg_hbm,
         acc_sh, src_v, dst_v, rows_v, sem) = refs
    c = lax.axis_index("c")
    s = lax.axis_index("s")
    wid = c * _NS + s

    r0 = pl.multiple_of(s * _RPT, 8)

    @pl.when(s < _ZT)
    def _zero_acc():
        pltpu.sync_copy(z2_hbm.at[pl.ds(r0, _RPT)], acc_sh.at[pl.ds(r0, _RPT)])
    if with_cnt:
        @pl.when(s < _CNT_T)
        def _zero_cnt():
            def zstep(i, carry):
                cnt_v[pl.ds(i * 16, 16)] = jnp.zeros((16,), jnp.float32)
                return carry
            lax.fori_loop(0, _CNT_R // 16, zstep, 0)
            q0 = pl.multiple_of(s * _CNT_R, 8)
            pltpu.sync_copy(cnt_v, cnt_sh.at[pl.ds(q0, _CNT_R)])
        for j in range(_K // 16):
            ones_v[pl.ds(j * 16, 16)] = jnp.ones((16,), jnp.float32)
    plsc.subcore_barrier()

    base = wid * _EPW

    def step(i, carry):
        e0 = pl.multiple_of(base + i * _K, 8)
        pltpu.sync_copy(src_hbm.at[pl.ds(e0, _K)], src_v)
        pltpu.sync_copy(dst_hbm.at[pl.ds(e0, _K)], dst_v)
        pltpu.async_copy(x_hbm.at[src_v], rows_v, sem).wait()
        pltpu.sync_copy(rows_v, acc_sh.at[dst_v], add=True)
        if with_cnt:
            pltpu.sync_copy(ones_v, cnt_sh.at[dst_v], add=True)
        return carry

    lax.fori_loop(0, _NIT, step, 0)
    plsc.subcore_barrier()

    @pl.when(s < _ZT)
    def _write_acc():
        pltpu.sync_copy(acc_sh.at[pl.ds(r0, _RPT)],
                        agg_hbm.at[c, pl.ds(r0, _RPT)])
    if with_cnt:
        @pl.when(s < _CNT_T)
        def _write_cnt():
            q0 = pl.multiple_of(s * _CNT_R, 8)
            qo = pl.multiple_of(c * _N + s * _CNT_R, 8)
            pltpu.sync_copy(cnt_sh.at[pl.ds(q0, _CNT_R)], cnt_v)
            pltpu.sync_copy(cnt_v, cnt_hbm.at[pl.ds(qo, _CNT_R)])


def _make_sc(with_cnt):
    mesh = plsc.VectorSubcoreMesh(core_axis_name="c", subcore_axis_name="s")
    if with_cnt:
        out_type = (jax.ShapeDtypeStruct((_NC, _N, _D), jnp.float32),
                    jax.ShapeDtypeStruct((_NC * _N,), jnp.float32))
        scratch = [
            pltpu.VMEM_SHARED((_N, _D), jnp.float32),
            pltpu.VMEM_SHARED((_N,), jnp.float32),
            pltpu.VMEM((_K,), jnp.int32),
            pltpu.VMEM((_K,), jnp.int32),
            pltpu.VMEM((_K, _D), jnp.float32),
            pltpu.VMEM((_K,), jnp.float32),
            pltpu.VMEM((_CNT_R,), jnp.float32),
            pltpu.SemaphoreType.DMA,
        ]
    else:
        out_type = jax.ShapeDtypeStruct((_NC, _N, _D), jnp.float32)
        scratch = [
            pltpu.VMEM_SHARED((_N, _D), jnp.float32),
            pltpu.VMEM((_K,), jnp.int32),
            pltpu.VMEM((_K,), jnp.int32),
            pltpu.VMEM((_K, _D), jnp.float32),
            pltpu.SemaphoreType.DMA,
        ]
    return pl.kernel(functools.partial(_sc_body, with_cnt),
                     out_type=out_type, mesh=mesh, scratch_types=scratch)


_B = 1000


def _tc_body(act, a0, a1, c0, c1, x, wl, bl, wr, o):
    deg = jnp.maximum(c0[...] + c1[...], 1.0)
    mean = (a0[...] + a1[...]) / deg
    y = (jnp.dot(mean, wl[...], preferred_element_type=jnp.float32)
         + bl[...]
         + jnp.dot(x[...], wr[...], preferred_element_type=jnp.float32))
    if act == "relu":
        o[...] = jnp.maximum(y, 0.0)
    else:
        o[...] = 1.0 / (1.0 + jnp.exp(-y))


def _make_tc(act):
    bs_r = pl.BlockSpec((_B, _D), lambda i: (i, 0))
    bs_c = pl.BlockSpec((_B, 1), lambda i: (i, 0))
    bs_w = pl.BlockSpec((_D, _D), lambda i: (0, 0))
    bs_b = pl.BlockSpec((1, _D), lambda i: (0, 0))
    return pl.pallas_call(
        functools.partial(_tc_body, act),
        grid=(_N // _B,),
        in_specs=[bs_r, bs_r, bs_c, bs_c, bs_r, bs_w, bs_b, bs_w],
        out_specs=bs_r,
        out_shape=jax.ShapeDtypeStruct((_N, _D), jnp.float32),
    )


def kernel(x, edge_index, Wl1, bl1, Wr1, Wl2, bl2, Wr2, Wl3, bl3, Wr3):
    src = edge_index[0]
    dst = edge_index[1]
    z2 = jnp.zeros((_N, _D), jnp.float32)

    sc_first = _make_sc(True)
    sc_rest = _make_sc(False)
    tc_relu = _make_tc("relu")
    tc_sig = _make_tc("sigmoid")

    agg, cnt = sc_first(x, src, dst, z2)
    cnt = cnt.reshape(_NC, _N)
    c0 = cnt[0].reshape(_N, 1)
    c1 = cnt[1].reshape(_N, 1)

    h = tc_relu(agg[0], agg[1], c0, c1, x,
                Wl1.T, bl1.reshape(1, _D), Wr1.T)
    agg2 = sc_rest(h, src, dst, z2)
    h2 = tc_relu(agg2[0], agg2[1], c0, c1, h,
                 Wl2.T, bl2.reshape(1, _D), Wr2.T)
    agg3 = sc_rest(h2, src, dst, z2)
    h3 = tc_sig(agg3[0], agg3[1], c0, c1, h2,
                Wl3.T, bl3.reshape(1, _D), Wr3.T)
    return h3


# speedup vs baseline: 8.5677x; 1.8226x over previous
import functools

import jax
import jax.numpy as jnp
from jax import lax
from jax.experimental import pallas as pl
from jax.experimental.pallas import tpu as pltpu
from jax.experimental.pallas import tpu_sc as plsc

_N = 10000
_E = 320000
_D = 128
_NC = 2
_NS = 16
_NW = _NC * _NS
_EPW = _E // _NW
_K = 80
_NIT = _EPW // _K
_ZT = 10
_RPT = _N // _ZT
_CNT_T = 5
_CNT_R = _N // _CNT_T


def _sc_body(with_cnt, *refs):
    if with_cnt:
        (x_hbm, src_hbm, dst_hbm, z2_hbm,
         agg_hbm, cnt_hbm,
         acc_sh, cnt_sh, src_a, dst_a, src_b, dst_b, rows_a, rows_b,
         ones_v, cnt_v, sem_ia, sem_ib, sem_a, sem_b) = refs
    else:
        (x_hbm, src_hbm, dst_hbm, z2_hbm,
         agg_hbm,
         acc_sh, src_a, dst_a, src_b, dst_b, rows_a, rows_b,
         sem_ia, sem_ib, sem_a, sem_b) = refs
    c = lax.axis_index("c")
    s = lax.axis_index("s")
    wid = c * _NS + s
    base = wid * _EPW

    def idxload(j, sv, dv, sem):
        e0 = pl.multiple_of(base + j * _K, 8)
        pltpu.async_copy(src_hbm.at[pl.ds(e0, _K)], sv, sem)
        pltpu.async_copy(dst_hbm.at[pl.ds(e0, _K)], dv, sem)

    def idxwait(sv, dv, sem):
        pltpu.make_async_copy(src_hbm.at[pl.ds(0, _K)], sv, sem).wait()
        pltpu.make_async_copy(src_hbm.at[pl.ds(0, _K)], dv, sem).wait()

    def gather(sv, buf, sem):
        return pltpu.async_copy(x_hbm.at[sv], buf, sem)

    def scatter(dv, buf):
        pltpu.sync_copy(buf, acc_sh.at[dv], add=True)
        if with_cnt:
            pltpu.sync_copy(ones_v, cnt_sh.at[dv], add=True)

    idxload(0, src_a, dst_a, sem_ia)

    r0 = pl.multiple_of(s * _RPT, 8)

    @pl.when(s < _ZT)
    def _zero_acc():
        pltpu.sync_copy(z2_hbm.at[pl.ds(r0, _RPT)], acc_sh.at[pl.ds(r0, _RPT)])
    if with_cnt:
        @pl.when(s < _CNT_T)
        def _zero_cnt():
            def zstep(i, carry):
                cnt_v[pl.ds(i * 16, 16)] = jnp.zeros((16,), jnp.float32)
                return carry
            lax.fori_loop(0, _CNT_R // 16, zstep, 0)
            q0 = pl.multiple_of(s * _CNT_R, 8)
            pltpu.sync_copy(cnt_v, cnt_sh.at[pl.ds(q0, _CNT_R)])
        for o in range(0, _K, 16):
            ones_v[pl.ds(o, 16)] = jnp.ones((16,), jnp.float32)
    idxwait(src_a, dst_a, sem_ia)
    cp_a0 = gather(src_a, rows_a, sem_a)
    idxload(1, src_b, dst_b, sem_ib)
    cp_a0.wait()
    plsc.subcore_barrier()

    def pair(ja, last):
        idxwait(src_b, dst_b, sem_ib)
        cp_b = gather(src_b, rows_b, sem_b)
        scatter(dst_a, rows_a)
        idxload(ja + 2, src_a, dst_a, sem_ia)
        cp_b.wait()
        idxwait(src_a, dst_a, sem_ia)
        cp_a = gather(src_a, rows_a, sem_a)
        scatter(dst_b, rows_b)
        if not last:
            idxload(ja + 3, src_b, dst_b, sem_ib)
        cp_a.wait()

    def step(j2, carry):
        pair(j2 * 2, False)
        return carry

    lax.fori_loop(0, (_NIT - 3) // 2, step, 0)
    pair(_NIT - 3, True)
    scatter(dst_a, rows_a)
    plsc.subcore_barrier()

    @pl.when(s < _ZT)
    def _write_acc():
        pltpu.sync_copy(acc_sh.at[pl.ds(r0, _RPT)],
                        agg_hbm.at[c, pl.ds(r0, _RPT)])
    if with_cnt:
        @pl.when(s < _CNT_T)
        def _write_cnt():
            q0 = pl.multiple_of(s * _CNT_R, 8)
            qo = pl.multiple_of(c * _N + s * _CNT_R, 8)
            pltpu.sync_copy(cnt_sh.at[pl.ds(q0, _CNT_R)], cnt_v)
            pltpu.sync_copy(cnt_v, cnt_hbm.at[pl.ds(qo, _CNT_R)])


def _make_sc(with_cnt):
    mesh = plsc.VectorSubcoreMesh(core_axis_name="c", subcore_axis_name="s")
    if with_cnt:
        out_type = (jax.ShapeDtypeStruct((_NC, _N, _D), jnp.float32),
                    jax.ShapeDtypeStruct((_NC * _N,), jnp.float32))
        scratch = [
            pltpu.VMEM_SHARED((_N, _D), jnp.float32),
            pltpu.VMEM_SHARED((_N,), jnp.float32),
            pltpu.VMEM((_K,), jnp.int32),
            pltpu.VMEM((_K,), jnp.int32),
            pltpu.VMEM((_K,), jnp.int32),
            pltpu.VMEM((_K,), jnp.int32),
            pltpu.VMEM((_K, _D), jnp.float32),
            pltpu.VMEM((_K, _D), jnp.float32),
            pltpu.VMEM((_K,), jnp.float32),
            pltpu.VMEM((_CNT_R,), jnp.float32),
            pltpu.SemaphoreType.DMA,
            pltpu.SemaphoreType.DMA,
            pltpu.SemaphoreType.DMA,
            pltpu.SemaphoreType.DMA,
        ]
    else:
        out_type = jax.ShapeDtypeStruct((_NC, _N, _D), jnp.float32)
        scratch = [
            pltpu.VMEM_SHARED((_N, _D), jnp.float32),
            pltpu.VMEM((_K,), jnp.int32),
            pltpu.VMEM((_K,), jnp.int32),
            pltpu.VMEM((_K,), jnp.int32),
            pltpu.VMEM((_K,), jnp.int32),
            pltpu.VMEM((_K, _D), jnp.float32),
            pltpu.VMEM((_K, _D), jnp.float32),
            pltpu.SemaphoreType.DMA,
            pltpu.SemaphoreType.DMA,
            pltpu.SemaphoreType.DMA,
            pltpu.SemaphoreType.DMA,
        ]
    return pl.kernel(functools.partial(_sc_body, with_cnt),
                     out_type=out_type, mesh=mesh, scratch_types=scratch)


_B = 1000


def _tc_body(act, a0, a1, c0, c1, x, wl, bl, wr, o):
    deg = jnp.maximum(c0[...] + c1[...], 1.0)
    mean = (a0[...] + a1[...]) / deg
    y = (jnp.dot(mean, wl[...], preferred_element_type=jnp.float32)
         + bl[...]
         + jnp.dot(x[...], wr[...], preferred_element_type=jnp.float32))
    if act == "relu":
        o[...] = jnp.maximum(y, 0.0)
    else:
        o[...] = 1.0 / (1.0 + jnp.exp(-y))


def _make_tc(act):
    bs_r = pl.BlockSpec((_B, _D), lambda i: (i, 0))
    bs_c = pl.BlockSpec((_B, 1), lambda i: (i, 0))
    bs_w = pl.BlockSpec((_D, _D), lambda i: (0, 0))
    bs_b = pl.BlockSpec((1, _D), lambda i: (0, 0))
    return pl.pallas_call(
        functools.partial(_tc_body, act),
        grid=(_N // _B,),
        in_specs=[bs_r, bs_r, bs_c, bs_c, bs_r, bs_w, bs_b, bs_w],
        out_specs=bs_r,
        out_shape=jax.ShapeDtypeStruct((_N, _D), jnp.float32),
    )


def kernel(x, edge_index, Wl1, bl1, Wr1, Wl2, bl2, Wr2, Wl3, bl3, Wr3):
    src = edge_index[0]
    dst = edge_index[1]
    z2 = jnp.zeros((_N, _D), jnp.float32)

    sc_first = _make_sc(True)
    sc_rest = _make_sc(False)
    tc_relu = _make_tc("relu")
    tc_sig = _make_tc("sigmoid")

    agg, cnt = sc_first(x, src, dst, z2)
    cnt = cnt.reshape(_NC, _N)
    c0 = cnt[0].reshape(_N, 1)
    c1 = cnt[1].reshape(_N, 1)

    h = tc_relu(agg[0], agg[1], c0, c1, x,
                Wl1.T, bl1.reshape(1, _D), Wr1.T)
    agg2 = sc_rest(h, src, dst, z2)
    h2 = tc_relu(agg2[0], agg2[1], c0, c1, h,
                 Wl2.T, bl2.reshape(1, _D), Wr2.T)
    agg3 = sc_rest(h2, src, dst, z2)
    h3 = tc_sig(agg3[0], agg3[1], c0, c1, h2,
                Wl3.T, bl3.reshape(1, _D), Wr3.T)
    return h3
